# Initial kernel scaffold; baseline (speedup 1.0000x reference)
import jax
import jax.numpy as jnp
from jax.experimental import pallas as pl


def kernel(m, edge_index, edge_vals):
    raise NotImplementedError("write your pallas kernel here")


# speedup vs baseline: 2.4132x; 2.4132x over previous
import jax
import jax.numpy as jnp
from jax import lax
from jax.experimental import pallas as pl
from jax.experimental.pallas import tpu as pltpu
from jax.experimental.pallas import tpu_sc as plsc

_N = 10000
_E = 320000
_D = 128

_NC = 2
_NS = 16
_HD = _D // _NC
_EPT = _E // _NS
_K = 80
_CH = _EPT // _K
_RPT = 624
_TAIL = _N - _NS * _RPT


def _sc_body(m_hbm, col_hbm, row_hbm, vals_hbm, zeros_hbm, out_hbm,
             colv, rowv, valv, grows, sbuf, acc, sem):
    cid = lax.axis_index("c")
    sid = lax.axis_index("s")

    pltpu.sync_copy(col_hbm.at[sid], colv)
    pltpu.sync_copy(row_hbm.at[sid], rowv)
    pltpu.sync_copy(zeros_hbm.at[pl.ds(sid * _RPT, _RPT)],
                    acc.at[pl.ds(sid * _RPT, _RPT)])

    @pl.when(sid == 0)
    def _():
        pltpu.sync_copy(zeros_hbm.at[pl.ds(_NS * _RPT, _TAIL)],
                        acc.at[pl.ds(_NS * _RPT, _TAIL)])

    base = jnp.full((16,), cid * _N, jnp.int32)

    def adj_body(i, carry):
        for r in range(_K // 16):
            sl = pl.ds(r * 16, 16)
            colv[i, sl] = colv[i, sl] + base
        return carry

    lax.fori_loop(0, _CH, adj_body, 0)
    plsc.subcore_barrier()

    def chunk_body(c, carry):
        pltpu.sync_copy(vals_hbm.at[sid, pl.ds(c * _K, _K)], valv)
        pltpu.async_copy(m_hbm.at[colv.at[c]], grows, sem).wait()

        def edge_body(e, carry2):
            vv = valv[e, :]
            for r in range(_HD // 16):
                sl = pl.ds(r * 16, 16)
                sbuf[e, sl] = grows[e, sl] * vv
            return carry2

        lax.fori_loop(0, _K, edge_body, 0)
        pltpu.sync_copy(sbuf, acc.at[rowv.at[c]], add=True)
        return carry

    lax.fori_loop(0, _CH, chunk_body, 0)
    plsc.subcore_barrier()
    pltpu.sync_copy(acc.at[pl.ds(sid * _RPT, _RPT)],
                    out_hbm.at[cid, pl.ds(sid * _RPT, _RPT)])

    @pl.when(sid == 0)
    def _():
        pltpu.sync_copy(acc.at[pl.ds(_NS * _RPT, _TAIL)],
                        out_hbm.at[cid, pl.ds(_NS * _RPT, _TAIL)])


_sc_spmm = pl.kernel(
    _sc_body,
    out_type=jax.ShapeDtypeStruct((_NC, _N, _HD), jnp.float32),
    mesh=plsc.VectorSubcoreMesh(core_axis_name="c", subcore_axis_name="s"),
    compiler_params=pltpu.CompilerParams(use_tc_tiling_on_sc=False),
    scratch_types=[
        pltpu.VMEM((_CH, _K), jnp.int32),
        pltpu.VMEM((_CH, _K), jnp.int32),
        pltpu.VMEM((_K, 16), jnp.float32),
        pltpu.VMEM((_K, _HD), jnp.float32),
        pltpu.VMEM((_K, _HD), jnp.float32),
        pltpu.VMEM_SHARED((_N, _HD), jnp.float32),
        pltpu.SemaphoreType.DMA,
    ],
)


def kernel(m, edge_index, edge_vals):
    row = edge_index[0].reshape(_NS, _CH, _K)
    col = edge_index[1].reshape(_NS, _CH, _K)
    vexp = jnp.broadcast_to(
        edge_vals.astype(jnp.float32)[:, None], (_E, 16)
    ).reshape(_NS, _EPT, 16)
    mf = m.astype(jnp.float32)
    m_split = jnp.concatenate([mf[:, :_HD], mf[:, _HD:]], axis=0)
    zeros = jnp.zeros((_N, _HD), jnp.float32)
    halves = _sc_spmm(m_split, col, row, vexp, zeros)
    out = jnp.swapaxes(halves, 0, 1).reshape(_N, _D)
    return out.astype(m.dtype)


# speedup vs baseline: 3.9722x; 1.6460x over previous
import jax
import jax.numpy as jnp
from jax import lax
from jax.experimental import pallas as pl
from jax.experimental.pallas import tpu as pltpu
from jax.experimental.pallas import tpu_sc as plsc

_N = 10000
_E = 320000
_D = 128

_NC = 2
_NS = 16
_HD = _D // _NC
_EPT = _E // _NS
_K = 80
_CH = _EPT // _K
_RPT = 624
_TAIL = _N - _NS * _RPT


def _sc_body(m_hbm, col_hbm, row_hbm, vals_hbm, zeros_hbm, out_hbm,
             colv, rowv, valv, grows, sbuf, acc, sg0, sg1, sv0, sv1):
    cid = lax.axis_index("c")
    sid = lax.axis_index("s")
    sg = (sg0, sg1)
    sv = (sv0, sv1)

    pltpu.sync_copy(col_hbm.at[sid], colv)
    pltpu.sync_copy(row_hbm.at[sid], rowv)
    pltpu.sync_copy(zeros_hbm.at[pl.ds(sid * _RPT, _RPT)],
                    acc.at[pl.ds(sid * _RPT, _RPT)])

    @pl.when(sid == 0)
    def _():
        pltpu.sync_copy(zeros_hbm.at[pl.ds(_NS * _RPT, _TAIL)],
                        acc.at[pl.ds(_NS * _RPT, _TAIL)])

    base = jnp.full((16,), cid * _N, jnp.int32)

    def adj_body(i, carry):
        for r in range(_K // 16):
            sl = pl.ds(r * 16, 16)
            colv[i, sl] = colv[i, sl] + base
        return carry

    lax.fori_loop(0, _CH, adj_body, 0)
    plsc.subcore_barrier()

    def _issue(c, b):
        pltpu.async_copy(vals_hbm.at[sid, pl.ds(c * _K, _K)], valv.at[b], sv[b])
        pltpu.async_copy(m_hbm.at[colv.at[c]], grows.at[b], sg[b])

    def _wait(c, b):
        pltpu.make_async_copy(
            vals_hbm.at[sid, pl.ds(c * _K, _K)], valv.at[b], sv[b]).wait()
        pltpu.make_async_copy(m_hbm.at[colv.at[c]], grows.at[b], sg[b]).wait()

    _issue(0, 0)
    _issue(1, 1)

    def chunk_pair(c2, carry):
        for b in range(2):
            c = 2 * c2 + b
            _wait(c, b)

            def edge_body(g, carry2):
                for u in range(4):
                    e = g * 4 + u
                    vv = valv[b, e, :]
                    for r in range(_HD // 16):
                        sl = pl.ds(r * 16, 16)
                        sbuf[e, sl] = grows[b, e, sl] * vv
                return carry2

            lax.fori_loop(0, _K // 4, edge_body, 0)
            pltpu.sync_copy(sbuf, acc.at[rowv.at[c]], add=True)

            @pl.when(c + 2 < _CH)
            def _():
                _issue(c + 2, b)

        return carry

    lax.fori_loop(0, _CH // 2, chunk_pair, 0)
    plsc.subcore_barrier()
    pltpu.sync_copy(acc.at[pl.ds(sid * _RPT, _RPT)],
                    out_hbm.at[cid, pl.ds(sid * _RPT, _RPT)])

    @pl.when(sid == 0)
    def _():
        pltpu.sync_copy(acc.at[pl.ds(_NS * _RPT, _TAIL)],
                        out_hbm.at[cid, pl.ds(_NS * _RPT, _TAIL)])


_sc_spmm = pl.kernel(
    _sc_body,
    out_type=jax.ShapeDtypeStruct((_NC, _N, _HD), jnp.float32),
    mesh=plsc.VectorSubcoreMesh(core_axis_name="c", subcore_axis_name="s"),
    compiler_params=pltpu.CompilerParams(use_tc_tiling_on_sc=False),
    scratch_types=[
        pltpu.VMEM((_CH, _K), jnp.int32),
        pltpu.VMEM((_CH, _K), jnp.int32),
        pltpu.VMEM((2, _K, 16), jnp.float32),
        pltpu.VMEM((2, _K, _HD), jnp.float32),
        pltpu.VMEM((_K, _HD), jnp.float32),
        pltpu.VMEM_SHARED((_N, _HD), jnp.float32),
        pltpu.SemaphoreType.DMA,
        pltpu.SemaphoreType.DMA,
        pltpu.SemaphoreType.DMA,
        pltpu.SemaphoreType.DMA,
    ],
)


def kernel(m, edge_index, edge_vals):
    row = edge_index[0].reshape(_NS, _CH, _K)
    col = edge_index[1].reshape(_NS, _CH, _K)
    vexp = jnp.broadcast_to(
        edge_vals.astype(jnp.float32)[:, None], (_E, 16)
    ).reshape(_NS, _EPT, 16)
    mf = m.astype(jnp.float32)
    m_split = jnp.concatenate([mf[:, :_HD], mf[:, _HD:]], axis=0)
    zeros = jnp.zeros((_N, _HD), jnp.float32)
    halves = _sc_spmm(m_split, col, row, vexp, zeros)
    out = jnp.swapaxes(halves, 0, 1).reshape(_N, _D)
    return out.astype(m.dtype)


# speedup vs baseline: 4.2219x; 1.0629x over previous
import jax
import jax.numpy as jnp
from jax import lax
from jax.experimental import pallas as pl
from jax.experimental.pallas import tpu as pltpu
from jax.experimental.pallas import tpu_sc as plsc

_N = 10000
_E = 320000
_D = 128

_NC = 2
_NS = 16
_HD = _D // _NC
_EPT = _E // _NS
_K = 80
_CH = _EPT // _K
_RPT = 624
_TAIL = _N - _NS * _RPT


def _sc_body(m_hbm, col_hbm, row_hbm, vals_hbm, zeros_hbm, out_hbm,
             colv, rowv, valv, grows, sbuf, acc, sg0, sg1, sv0, sv1, ss0, ss1):
    cid = lax.axis_index("c")
    sid = lax.axis_index("s")
    sg = (sg0, sg1)
    sv = (sv0, sv1)
    ss = (ss0, ss1)

    pltpu.sync_copy(col_hbm.at[sid], colv)
    pltpu.sync_copy(row_hbm.at[sid], rowv)
    pltpu.sync_copy(zeros_hbm.at[pl.ds(sid * _RPT, _RPT)],
                    acc.at[pl.ds(sid * _RPT, _RPT)])

    @pl.when(sid == 0)
    def _():
        pltpu.sync_copy(zeros_hbm.at[pl.ds(_NS * _RPT, _TAIL)],
                        acc.at[pl.ds(_NS * _RPT, _TAIL)])

    base = jnp.full((16,), cid * _N, jnp.int32)

    def adj_body(i, carry):
        for r in range(_K // 16):
            sl = pl.ds(r * 16, 16)
            colv[i, sl] = colv[i, sl] + base
        return carry

    lax.fori_loop(0, _CH, adj_body, 0)
    plsc.subcore_barrier()

    def _issue(c, b):
        pltpu.async_copy(vals_hbm.at[sid, pl.ds(c * _K, _K)], valv.at[b], sv[b])
        pltpu.async_copy(m_hbm.at[colv.at[c]], grows.at[b], sg[b])

    def _wait(c, b):
        pltpu.make_async_copy(
            vals_hbm.at[sid, pl.ds(c * _K, _K)], valv.at[b], sv[b]).wait()
        pltpu.make_async_copy(m_hbm.at[colv.at[c]], grows.at[b], sg[b]).wait()

    _issue(0, 0)
    _issue(1, 1)

    def chunk_pair(c2, carry):
        for b in range(2):
            c = 2 * c2 + b
            _wait(c, b)

            @pl.when(c2 >= 1)
            def _():
                pltpu.make_async_copy(
                    sbuf.at[b], acc.at[rowv.at[c]], ss[b]).wait()

            def edge_body(g, carry2):
                for u in range(4):
                    e = g * 4 + u
                    vv = valv[b, e, :]
                    for r in range(_HD // 16):
                        sl = pl.ds(r * 16, 16)
                        sbuf[b, e, sl] = grows[b, e, sl] * vv
                return carry2

            lax.fori_loop(0, _K // 4, edge_body, 0)
            pltpu.async_copy(sbuf.at[b], acc.at[rowv.at[c]], ss[b], add=True)

            @pl.when(c + 2 < _CH)
            def _():
                _issue(c + 2, b)

        return carry

    lax.fori_loop(0, _CH // 2, chunk_pair, 0)
    for b in range(2):
        pltpu.make_async_copy(sbuf.at[b], acc.at[rowv.at[b]], ss[b]).wait()
    plsc.subcore_barrier()
    pltpu.sync_copy(acc.at[pl.ds(sid * _RPT, _RPT)],
                    out_hbm.at[cid, pl.ds(sid * _RPT, _RPT)])

    @pl.when(sid == 0)
    def _():
        pltpu.sync_copy(acc.at[pl.ds(_NS * _RPT, _TAIL)],
                        out_hbm.at[cid, pl.ds(_NS * _RPT, _TAIL)])


_sc_spmm = pl.kernel(
    _sc_body,
    out_type=jax.ShapeDtypeStruct((_NC, _N, _HD), jnp.float32),
    mesh=plsc.VectorSubcoreMesh(core_axis_name="c", subcore_axis_name="s"),
    compiler_params=pltpu.CompilerParams(use_tc_tiling_on_sc=False),
    scratch_types=[
        pltpu.VMEM((_CH, _K), jnp.int32),
        pltpu.VMEM((_CH, _K), jnp.int32),
        pltpu.VMEM((2, _K, 16), jnp.float32),
        pltpu.VMEM((2, _K, _HD), jnp.float32),
        pltpu.VMEM((2, _K, _HD), jnp.float32),
        pltpu.VMEM_SHARED((_N, _HD), jnp.float32),
        pltpu.SemaphoreType.DMA,
        pltpu.SemaphoreType.DMA,
        pltpu.SemaphoreType.DMA,
        pltpu.SemaphoreType.DMA,
        pltpu.SemaphoreType.DMA,
        pltpu.SemaphoreType.DMA,
    ],
)


def kernel(m, edge_index, edge_vals):
    row = edge_index[0].reshape(_NS, _CH, _K)
    col = edge_index[1].reshape(_NS, _CH, _K)
    vexp = jnp.broadcast_to(
        edge_vals.astype(jnp.float32)[:, None], (_E, 16)
    ).reshape(_NS, _EPT, 16)
    mf = m.astype(jnp.float32)
    m_split = jnp.concatenate([mf[:, :_HD], mf[:, _HD:]], axis=0)
    zeros = jnp.zeros((_N, _HD), jnp.float32)
    halves = _sc_spmm(m_split, col, row, vexp, zeros)
    out = jnp.swapaxes(halves, 0, 1).reshape(_N, _D)
    return out.astype(m.dtype)


# speedup vs baseline: 6.9207x; 1.6392x over previous
import jax
import jax.numpy as jnp
from jax import lax
from jax.experimental import pallas as pl
from jax.experimental.pallas import tpu as pltpu
from jax.experimental.pallas import tpu_sc as plsc

_N = 10000
_E = 320000
_D = 128

_NC = 2
_NS = 16
_HD = _D // _NC
_EPT = _E // _NS
_K = 80
_KG = _K // 8
_CH = _EPT // _K
_RPT = 624
_TAIL = _N - _NS * _RPT


def _sc_body(m_hbm, col_hbm, row_hbm, vals_hbm, zeros_hbm, out_hbm,
             colv, rowv, valv, grows, sbuf, acc, sg0, sg1, sv0, sv1, ss0, ss1):
    cid = lax.axis_index("c")
    sid = lax.axis_index("s")
    sg = (sg0, sg1)
    sv = (sv0, sv1)
    ss = (ss0, ss1)

    pltpu.sync_copy(col_hbm.at[sid], colv)
    pltpu.sync_copy(row_hbm.at[sid], rowv)
    pltpu.sync_copy(zeros_hbm.at[pl.ds(sid * _RPT, _RPT)],
                    acc.at[pl.ds(sid * _RPT, _RPT)])

    @pl.when(sid == 0)
    def _():
        pltpu.sync_copy(zeros_hbm.at[pl.ds(_NS * _RPT, _TAIL)],
                        acc.at[pl.ds(_NS * _RPT, _TAIL)])

    base = jnp.full((16,), cid, jnp.int32)

    def adj_body(i, carry):
        for r in range(_K // 16):
            sl = pl.ds(r * 16, 16)
            cv = colv[i, sl]
            colv[i, sl] = cv + cv + base
        return carry

    lax.fori_loop(0, _CH, adj_body, 0)
    plsc.subcore_barrier()

    def _issue(c, b):
        pltpu.async_copy(vals_hbm.at[sid, pl.ds(c * _KG, _KG)], valv.at[b], sv[b])
        pltpu.async_copy(m_hbm.at[colv.at[c]], grows.at[b], sg[b])

    def _wait(c, b):
        pltpu.make_async_copy(
            vals_hbm.at[sid, pl.ds(c * _KG, _KG)], valv.at[b], sv[b]).wait()
        pltpu.make_async_copy(m_hbm.at[colv.at[c]], grows.at[b], sg[b]).wait()

    _issue(0, 0)
    _issue(1, 1)

    def chunk_pair(c2, carry):
        for b in range(2):
            c = 2 * c2 + b
            _wait(c, b)

            @pl.when(c2 >= 1)
            def _():
                pltpu.make_async_copy(
                    sbuf.at[b], acc.at[rowv.at[c]], ss[b]).wait()

            def edge_body(g, carry2):
                for u in range(8):
                    e = g * 8 + u
                    vv = valv[b, g, pl.ds(u * 16, 16)]
                    for r in range(_HD // 16):
                        sl = pl.ds(r * 16, 16)
                        sbuf[b, e, sl] = grows[b, e, sl] * vv
                return carry2

            lax.fori_loop(0, _KG, edge_body, 0)
            pltpu.async_copy(sbuf.at[b], acc.at[rowv.at[c]], ss[b], add=True)

            @pl.when(c + 2 < _CH)
            def _():
                _issue(c + 2, b)

        return carry

    lax.fori_loop(0, _CH // 2, chunk_pair, 0)
    for b in range(2):
        pltpu.make_async_copy(sbuf.at[b], acc.at[rowv.at[b]], ss[b]).wait()
    plsc.subcore_barrier()
    pltpu.sync_copy(acc.at[pl.ds(sid * _RPT, _RPT)],
                    out_hbm.at[cid, pl.ds(sid * _RPT, _RPT)])

    @pl.when(sid == 0)
    def _():
        pltpu.sync_copy(acc.at[pl.ds(_NS * _RPT, _TAIL)],
                        out_hbm.at[cid, pl.ds(_NS * _RPT, _TAIL)])


_sc_spmm = pl.kernel(
    _sc_body,
    out_type=jax.ShapeDtypeStruct((_NC, _N, _HD), jnp.float32),
    mesh=plsc.VectorSubcoreMesh(core_axis_name="c", subcore_axis_name="s"),
    compiler_params=pltpu.CompilerParams(use_tc_tiling_on_sc=False),
    scratch_types=[
        pltpu.VMEM((_CH, _K), jnp.int32),
        pltpu.VMEM((_CH, _K), jnp.int32),
        pltpu.VMEM((2, _KG, 128), jnp.float32),
        pltpu.VMEM((2, _K, _HD), jnp.float32),
        pltpu.VMEM((2, _K, _HD), jnp.float32),
        pltpu.VMEM_SHARED((_N, _HD), jnp.float32),
        pltpu.SemaphoreType.DMA,
        pltpu.SemaphoreType.DMA,
        pltpu.SemaphoreType.DMA,
        pltpu.SemaphoreType.DMA,
        pltpu.SemaphoreType.DMA,
        pltpu.SemaphoreType.DMA,
    ],
)


def kernel(m, edge_index, edge_vals):
    row = edge_index[0].reshape(_NS, _CH, _K)
    col = edge_index[1].reshape(_NS, _CH, _K)
    vexp = jnp.broadcast_to(
        edge_vals.astype(jnp.float32)[:, None], (_E, 16)
    ).reshape(_NS, _EPT // 8, 128)
    m_split = m.astype(jnp.float32).reshape(2 * _N, _HD)
    zeros = jnp.zeros((_N, _HD), jnp.float32)
    halves = _sc_spmm(m_split, col, row, vexp, zeros)
    out = jnp.swapaxes(halves, 0, 1).reshape(_N, _D)
    return out.astype(m.dtype)


# speedup vs baseline: 7.7084x; 1.1138x over previous
import jax
import jax.numpy as jnp
from jax import lax
from jax.experimental import pallas as pl
from jax.experimental.pallas import tpu as pltpu
from jax.experimental.pallas import tpu_sc as plsc

_N = 10000
_E = 320000
_D = 128

_NC = 2
_NS = 16
_HD = _D // _NC
_EPT = _E // _NS
_K = 80
_KG = _K // 8
_CH = _EPT // _K
_RPT = 624
_TAIL = _N - _NS * _RPT


def _sc_body(m_hbm, col_hbm, row_hbm, vals_hbm, zeros_hbm, out_hbm,
             colv, rowv, valv, grows, sbuf, acc, sg0, sg1, sv0, sv1, ss0, ss1):
    cid = lax.axis_index("c")
    sid = lax.axis_index("s")
    sg = (sg0, sg1)
    sv = (sv0, sv1)
    ss = (ss0, ss1)

    pltpu.sync_copy(col_hbm.at[sid], colv)
    pltpu.sync_copy(row_hbm.at[sid], rowv)
    pltpu.sync_copy(zeros_hbm.at[pl.ds(sid * _RPT, _RPT)],
                    acc.at[pl.ds(sid * _RPT, _RPT)])

    @pl.when(sid == 0)
    def _():
        pltpu.sync_copy(zeros_hbm.at[pl.ds(_NS * _RPT, _TAIL)],
                        acc.at[pl.ds(_NS * _RPT, _TAIL)])

    base = jnp.full((16,), cid, jnp.int32)

    def adj_body(i, carry):
        for r in range(_K // 16):
            sl = pl.ds(r * 16, 16)
            cv = colv[i, sl]
            colv[i, sl] = cv + cv + base
        return carry

    lax.fori_loop(0, _CH, adj_body, 0)
    plsc.subcore_barrier()

    def _issue(c, b):
        pltpu.async_copy(vals_hbm.at[sid, pl.ds(c * _KG, _KG)], valv.at[b], sv[b])
        pltpu.async_copy(m_hbm.at[colv.at[c]], grows.at[b], sg[b])

    def _wait(c, b):
        pltpu.make_async_copy(
            vals_hbm.at[sid, pl.ds(c * _KG, _KG)], valv.at[b], sv[b]).wait()
        pltpu.make_async_copy(m_hbm.at[colv.at[c]], grows.at[b], sg[b]).wait()

    _issue(0, 0)
    _issue(1, 1)

    def chunk_pair(c2, carry):
        for b in range(2):
            c = 2 * c2 + b
            _wait(c, b)

            @pl.when(c2 >= 1)
            def _():
                pltpu.make_async_copy(
                    sbuf.at[b], acc.at[rowv.at[c]], ss[b]).wait()

            def edge_body(g, carry2):
                nr = _HD // 16
                vvs = [valv[b, g, pl.ds(u * 16, 16)] for u in range(8)]
                gvs = [[grows[b, g * 8 + u, pl.ds(r * 16, 16)]
                        for r in range(nr)] for u in range(8)]
                for u in range(8):
                    for r in range(nr):
                        sbuf[b, g * 8 + u, pl.ds(r * 16, 16)] = gvs[u][r] * vvs[u]
                return carry2

            lax.fori_loop(0, _KG, edge_body, 0)
            pltpu.async_copy(sbuf.at[b], acc.at[rowv.at[c]], ss[b], add=True)

            @pl.when(c + 2 < _CH)
            def _():
                _issue(c + 2, b)

        return carry

    lax.fori_loop(0, _CH // 2, chunk_pair, 0)
    for b in range(2):
        pltpu.make_async_copy(sbuf.at[b], acc.at[rowv.at[b]], ss[b]).wait()
    plsc.subcore_barrier()
    pltpu.sync_copy(acc.at[pl.ds(sid * _RPT, _RPT)],
                    out_hbm.at[cid, pl.ds(sid * _RPT, _RPT)])

    @pl.when(sid == 0)
    def _():
        pltpu.sync_copy(acc.at[pl.ds(_NS * _RPT, _TAIL)],
                        out_hbm.at[cid, pl.ds(_NS * _RPT, _TAIL)])


_sc_spmm = pl.kernel(
    _sc_body,
    out_type=jax.ShapeDtypeStruct((_NC, _N, _HD), jnp.float32),
    mesh=plsc.VectorSubcoreMesh(core_axis_name="c", subcore_axis_name="s"),
    compiler_params=pltpu.CompilerParams(use_tc_tiling_on_sc=False),
    scratch_types=[
        pltpu.VMEM((_CH, _K), jnp.int32),
        pltpu.VMEM((_CH, _K), jnp.int32),
        pltpu.VMEM((2, _KG, 128), jnp.float32),
        pltpu.VMEM((2, _K, _HD), jnp.float32),
        pltpu.VMEM((2, _K, _HD), jnp.float32),
        pltpu.VMEM_SHARED((_N, _HD), jnp.float32),
        pltpu.SemaphoreType.DMA,
        pltpu.SemaphoreType.DMA,
        pltpu.SemaphoreType.DMA,
        pltpu.SemaphoreType.DMA,
        pltpu.SemaphoreType.DMA,
        pltpu.SemaphoreType.DMA,
    ],
)


def kernel(m, edge_index, edge_vals):
    row = edge_index[0].reshape(_NS, _CH, _K)
    col = edge_index[1].reshape(_NS, _CH, _K)
    vexp = jnp.broadcast_to(
        edge_vals.astype(jnp.float32)[:, None], (_E, 16)
    ).reshape(_NS, _EPT // 8, 128)
    m_split = m.astype(jnp.float32).reshape(2 * _N, _HD)
    zeros = jnp.zeros((_N, _HD), jnp.float32)
    halves = _sc_spmm(m_split, col, row, vexp, zeros)
    out = jnp.swapaxes(halves, 0, 1).reshape(_N, _D)
    return out.astype(m.dtype)


# speedup vs baseline: 10.0050x; 1.2979x over previous
import jax
import jax.numpy as jnp
from jax import lax
from jax.experimental import pallas as pl
from jax.experimental.pallas import tpu as pltpu
from jax.experimental.pallas import tpu_sc as plsc

_N = 10000
_E = 320000
_D = 128

_NC = 2
_NS = 16
_HD = _D // _NC
_EPT = _E // _NS
_K = 80
_KG = _K // 8
_CH = _EPT // _K
_RPT = 624
_TAIL = _N - _NS * _RPT


def _sc_body(m_hbm, col_hbm, row_hbm, vals_hbm, zeros_hbm, out_hbm,
             colv, rowv, valv, grows, sbuf, acc, sg0, sg1, ss0, ss1):
    cid = lax.axis_index("c")
    sid = lax.axis_index("s")
    sg = (sg0, sg1)
    ss = (ss0, ss1)

    pltpu.sync_copy(col_hbm.at[sid], colv)
    pltpu.sync_copy(row_hbm.at[sid], rowv)
    pltpu.sync_copy(vals_hbm.at[pl.ds(sid * _EPT, _EPT)], valv)
    pltpu.sync_copy(zeros_hbm.at[pl.ds(sid * _RPT, _RPT)],
                    acc.at[pl.ds(sid * _RPT, _RPT)])

    @pl.when(sid == 0)
    def _():
        pltpu.sync_copy(zeros_hbm.at[pl.ds(_NS * _RPT, _TAIL)],
                        acc.at[pl.ds(_NS * _RPT, _TAIL)])

    base = jnp.full((16,), cid, jnp.int32)

    def adj_body(i, carry):
        for r in range(_K // 16):
            sl = pl.ds(r * 16, 16)
            cv = colv[i, sl]
            colv[i, sl] = cv + cv + base
        return carry

    lax.fori_loop(0, _CH, adj_body, 0)
    plsc.subcore_barrier()

    def _issue(c, b):
        pltpu.async_copy(m_hbm.at[colv.at[c]], grows.at[b], sg[b])

    def _wait(c, b):
        pltpu.make_async_copy(m_hbm.at[colv.at[c]], grows.at[b], sg[b]).wait()

    _issue(0, 0)
    _issue(1, 1)

    def chunk_pair(c2, carry):
        for b in range(2):
            c = 2 * c2 + b
            _wait(c, b)

            @pl.when(c2 >= 1)
            def _():
                pltpu.make_async_copy(
                    sbuf.at[b], acc.at[rowv.at[c]], ss[b]).wait()

            def edge_body(g, carry2):
                nr = _HD // 16
                vals16 = valv[pl.ds(c * _K + g * 16, 16)]
                dnums = lax.GatherDimensionNumbers(
                    offset_dims=(), collapsed_slice_dims=(0,),
                    start_index_map=(0,))
                for h in range(2):
                    vvs = [lax.gather(
                        vals16, jnp.full((16, 1), h * 8 + u, jnp.int32),
                        dnums, slice_sizes=(1,),
                        mode=lax.GatherScatterMode.PROMISE_IN_BOUNDS)
                           for u in range(8)]
                    gvs = [[grows[b, g * 16 + h * 8 + u, pl.ds(r * 16, 16)]
                            for r in range(nr)] for u in range(8)]
                    for u in range(8):
                        for r in range(nr):
                            sbuf[b, g * 16 + h * 8 + u, pl.ds(r * 16, 16)] = (
                                gvs[u][r] * vvs[u])
                return carry2

            lax.fori_loop(0, _K // 16, edge_body, 0)
            pltpu.async_copy(sbuf.at[b], acc.at[rowv.at[c]], ss[b], add=True)

            @pl.when(c + 2 < _CH)
            def _():
                _issue(c + 2, b)

        return carry

    lax.fori_loop(0, _CH // 2, chunk_pair, 0)
    for b in range(2):
        pltpu.make_async_copy(sbuf.at[b], acc.at[rowv.at[b]], ss[b]).wait()
    plsc.subcore_barrier()
    pltpu.sync_copy(acc.at[pl.ds(sid * _RPT, _RPT)],
                    out_hbm.at[cid, pl.ds(sid * _RPT, _RPT)])

    @pl.when(sid == 0)
    def _():
        pltpu.sync_copy(acc.at[pl.ds(_NS * _RPT, _TAIL)],
                        out_hbm.at[cid, pl.ds(_NS * _RPT, _TAIL)])


_sc_spmm = pl.kernel(
    _sc_body,
    out_type=jax.ShapeDtypeStruct((_NC, _N, _HD), jnp.float32),
    mesh=plsc.VectorSubcoreMesh(core_axis_name="c", subcore_axis_name="s"),
    compiler_params=pltpu.CompilerParams(use_tc_tiling_on_sc=False),
    scratch_types=[
        pltpu.VMEM((_CH, _K), jnp.int32),
        pltpu.VMEM((_CH, _K), jnp.int32),
        pltpu.VMEM((_EPT,), jnp.float32),
        pltpu.VMEM((2, _K, _HD), jnp.float32),
        pltpu.VMEM((2, _K, _HD), jnp.float32),
        pltpu.VMEM_SHARED((_N, _HD), jnp.float32),
        pltpu.SemaphoreType.DMA,
        pltpu.SemaphoreType.DMA,
        pltpu.SemaphoreType.DMA,
        pltpu.SemaphoreType.DMA,
    ],
)


def kernel(m, edge_index, edge_vals):
    row = edge_index[0].reshape(_NS, _CH, _K)
    col = edge_index[1].reshape(_NS, _CH, _K)
    m_split = m.astype(jnp.float32).reshape(2 * _N, _HD)
    zeros = jnp.zeros((_N, _HD), jnp.float32)
    halves = _sc_spmm(m_split, col, row, edge_vals.astype(jnp.float32),
                      zeros)
    out = jnp.swapaxes(halves, 0, 1).reshape(_N, _D)
    return out.astype(m.dtype)
